# baseline (device time: 218601 ns/iter reference)
import functools

import jax
import jax.numpy as jnp
from jax import lax
from jax.experimental import pallas as pl
from jax.experimental.pallas import tpu as pltpu

N_DEV = 32
M, N = 4096, 2048
CH = M // N_DEV
HN = N // 2
NS = 4
SC = 8
RH = CH // SC


def _ring_tables():
    mesh_order = []
    for z in range(4):
        for y in range(4):
            xs = (0, 1) if y % 2 == 0 else (1, 0)
            for x in xs:
                mesh_order.append((x, y, z))
    pos_of = {c: i for i, c in enumerate(mesh_order)}

    yz_path = []
    for y in range(4):
        zs = range(4) if y % 2 == 0 else range(3, -1, -1)
        for z in zs:
            yz_path.append((y, z))
    cycle = [(0, y, z) for (y, z) in yz_path]
    cycle += [(1, y, z) for (y, z) in reversed(yz_path)]
    for a, b in zip(cycle, cycle[1:] + cycle[:1]):
        assert sum(abs(p - q) for p, q in zip(a, b)) == 1, (a, b)

    k_of_pos = [0] * N_DEV
    next_of_pos = [0] * N_DEV
    prev_of_pos = [0] * N_DEV
    for k, c in enumerate(cycle):
        p = pos_of[c]
        k_of_pos[p] = k
        next_of_pos[p] = pos_of[cycle[(k + 1) % N_DEV]]
        prev_of_pos[p] = pos_of[cycle[(k - 1) % N_DEV]]
    return k_of_pos, next_of_pos, prev_of_pos


_K_OF_POS, _NEXT_OF_POS, _PREV_OF_POS = _ring_tables()


def kernel(x, w_mat):
    partial = jnp.dot(
        x.astype(jnp.bfloat16),
        w_mat.astype(jnp.bfloat16),
        preferred_element_type=jnp.float32,
    ).astype(jnp.bfloat16)
    me = lax.axis_index("i")
    scalars = jnp.stack([
        jnp.asarray(_K_OF_POS, jnp.int32)[me],
        jnp.asarray(_NEXT_OF_POS, jnp.int32)[me],
        jnp.asarray(_PREV_OF_POS, jnp.int32)[me],
    ])
    return _all_reduce_relu(scalars, partial)


def _all_reduce_relu(scalars, partial):
    def body(sc_ref, p_ref, out_ref, comm_ref, local_ref, ostage_ref,
             send_sems, recv_sems, local_sems, out_sems, stage_sems):
        k = sc_ref[0]
        nxt = sc_ref[1]
        prv = sc_ref[2]

        dsts = (nxt, prv)
        sgns = (-1, 1)
        col0s = (0, HN)

        def rows(j):
            return pl.ds(j * RH, RH)

        def hop_rdma(r, t, j):
            ss, rs = t % NS, (t + 1) % NS
            return pltpu.make_async_remote_copy(
                src_ref=comm_ref.at[r, ss, rows(j)],
                dst_ref=comm_ref.at[r, rs, rows(j)],
                send_sem=send_sems.at[r, ss, j],
                recv_sem=recv_sems.at[r, rs, j],
                device_id=(dsts[r],),
                device_id_type=pl.DeviceIdType.MESH,
            )

        send_pending = {}

        def start_send(r, t, j):
            key = (r, t % NS, j)
            prev = send_pending.get(key)
            if prev is not None:
                prev.wait_send()
            d = hop_rdma(r, t, j)
            d.start()
            send_pending[key] = d

        def local_dma(r, s):
            c = jnp.mod(k + sgns[r] * (s + 1), N_DEV)
            return pltpu.make_async_copy(
                p_ref.at[pl.ds(c * CH, CH), pl.ds(col0s[r], HN)],
                local_ref.at[r, s % 2], local_sems.at[r, s % 2])

        own = []
        for r in range(2):
            dma = pltpu.make_async_copy(
                p_ref.at[pl.ds(k * CH, CH), pl.ds(col0s[r], HN)],
                comm_ref.at[r, 0], stage_sems.at[r])
            dma.start()
            own.append(dma)
        ldma = [local_dma(0, 0), local_dma(1, 0)]
        for d in ldma:
            d.start()

        barrier_sem = pltpu.get_barrier_semaphore()
        for nbr in (nxt, prv):
            pl.semaphore_signal(barrier_sem, inc=1, device_id=(nbr,),
                                device_id_type=pl.DeviceIdType.MESH)
        pl.semaphore_wait(barrier_sem, 2)
        for dma in own:
            dma.wait()

        for j in range(SC):
            for r in range(2):
                start_send(r, 0, j)

        out_dma = [[None, None], [None, None]]

        for s in range(N_DEV - 1):
            rs_slot = (s + 1) % NS
            nldma = None
            if s + 1 < N_DEV - 1:
                nldma = [local_dma(0, s + 1), local_dma(1, s + 1)]
                for d in nldma:
                    d.start()
            for r in range(2):
                ldma[r].wait()
            for j in range(SC):
                for r in range(2):
                    hop_rdma(r, s, j).wait_recv()
                    acc = (comm_ref[r, rs_slot, j * RH:(j + 1) * RH]
                           .astype(jnp.float32)
                           + local_ref[r, s % 2, j * RH:(j + 1) * RH]
                           .astype(jnp.float32))
                    if s < N_DEV - 2:
                        comm_ref[r, rs_slot, j * RH:(j + 1) * RH] = (
                            acc.astype(jnp.bfloat16))
                        start_send(r, s + 1, j)
                    else:
                        rr = jnp.maximum(acc, 0.0)
                        comm_ref[r, rs_slot, j * RH:(j + 1) * RH] = (
                            rr.astype(jnp.bfloat16))
                        start_send(r, N_DEV - 1, j)
                        ostage_ref[r, 0, j * RH:(j + 1) * RH] = rr
            ldma = nldma
            if s == N_DEV - 2:
                for r in range(2):
                    c_own = jnp.mod(k - sgns[r], N_DEV)
                    odma = pltpu.make_async_copy(
                        ostage_ref.at[r, 0],
                        out_ref.at[pl.ds(c_own * CH, CH), pl.ds(col0s[r], HN)],
                        out_sems.at[r, 0])
                    odma.start()
                    out_dma[r][0] = odma

        for h in range(N_DEV - 1):
            t = N_DEV - 1 + h
            rs_slot = (t + 1) % NS
            for j in range(SC):
                for r in range(2):
                    hop_rdma(r, t, j).wait_recv()
                    if h < N_DEV - 2:
                        start_send(r, t + 1, j)
            oslot = h % 2
            for r in range(2):
                c = jnp.mod(k + sgns[r] * h, N_DEV)
                if out_dma[r][oslot] is not None:
                    out_dma[r][oslot].wait()
                ostage_ref[r, oslot] = comm_ref[r, rs_slot].astype(jnp.float32)
                odma = pltpu.make_async_copy(
                    ostage_ref.at[r, oslot],
                    out_ref.at[pl.ds(c * CH, CH), pl.ds(col0s[r], HN)],
                    out_sems.at[r, oslot])
                odma.start()
                out_dma[r][oslot] = odma

        for r in range(2):
            for sl in range(2):
                out_dma[r][sl].wait()
        for d in send_pending.values():
            d.wait_send()

        @functools.partial(pl.run_scoped, sem2=pltpu.SemaphoreType.REGULAR)
        def _(sem2):
            for nbr in (nxt, prv):
                pl.semaphore_signal(sem2, inc=1, device_id=(nbr,),
                                    device_id_type=pl.DeviceIdType.MESH)
            pl.semaphore_wait(sem2, 2)

    return pl.pallas_call(
        body,
        out_shape=jax.ShapeDtypeStruct((M, N), jnp.float32),
        in_specs=[
            pl.BlockSpec(memory_space=pltpu.SMEM),
            pl.BlockSpec(memory_space=pl.ANY),
        ],
        out_specs=pl.BlockSpec(memory_space=pl.ANY),
        scratch_shapes=[
            pltpu.VMEM((2, NS, CH, HN), jnp.bfloat16),
            pltpu.VMEM((2, 2, CH, HN), jnp.bfloat16),
            pltpu.VMEM((2, 2, CH, HN), jnp.float32),
            pltpu.SemaphoreType.DMA((2, NS, SC)),
            pltpu.SemaphoreType.DMA((2, NS, SC)),
            pltpu.SemaphoreType.DMA((2, 2)),
            pltpu.SemaphoreType.DMA((2, 2)),
            pltpu.SemaphoreType.DMA((2,)),
        ],
        compiler_params=pltpu.CompilerParams(collective_id=0),
    )(scalars, partial)


# device time: 215086 ns/iter; 1.0163x vs baseline; 1.0163x over previous
import functools

import jax
import jax.numpy as jnp
from jax import lax
from jax.experimental import pallas as pl
from jax.experimental.pallas import tpu as pltpu

N_DEV = 32
M, N = 4096, 2048
K_SH = 128
CH = M // N_DEV
HN = N // 2
NS = 4
SC = 4
RH = CH // SC


def _ring_tables():
    mesh_order = []
    for z in range(4):
        for y in range(4):
            xs = (0, 1) if y % 2 == 0 else (1, 0)
            for x in xs:
                mesh_order.append((x, y, z))
    pos_of = {c: i for i, c in enumerate(mesh_order)}

    yz_path = []
    for y in range(4):
        zs = range(4) if y % 2 == 0 else range(3, -1, -1)
        for z in zs:
            yz_path.append((y, z))
    cycle = [(0, y, z) for (y, z) in yz_path]
    cycle += [(1, y, z) for (y, z) in reversed(yz_path)]
    for a, b in zip(cycle, cycle[1:] + cycle[:1]):
        assert sum(abs(p - q) for p, q in zip(a, b)) == 1, (a, b)

    k_of_pos = [0] * N_DEV
    next_of_pos = [0] * N_DEV
    prev_of_pos = [0] * N_DEV
    for k, c in enumerate(cycle):
        p = pos_of[c]
        k_of_pos[p] = k
        next_of_pos[p] = pos_of[cycle[(k + 1) % N_DEV]]
        prev_of_pos[p] = pos_of[cycle[(k - 1) % N_DEV]]
    return k_of_pos, next_of_pos, prev_of_pos


_K_OF_POS, _NEXT_OF_POS, _PREV_OF_POS = _ring_tables()


def kernel(x, w_mat):
    me = lax.axis_index("i")
    scalars = jnp.stack([
        jnp.asarray(_K_OF_POS, jnp.int32)[me],
        jnp.asarray(_NEXT_OF_POS, jnp.int32)[me],
        jnp.asarray(_PREV_OF_POS, jnp.int32)[me],
    ])
    x3 = x.astype(jnp.bfloat16).reshape(N_DEV, CH, K_SH)
    w = w_mat.astype(jnp.bfloat16)
    return _gemm_all_reduce_relu(scalars, x3, w)


def _gemm_all_reduce_relu(scalars, x3, w):
    def body(sc_ref, x_ref, w_ref, out_ref, comm_ref, local_ref, ostage_ref,
             send_sems, recv_sems, out_sems):
        k = sc_ref[0]
        nxt = sc_ref[1]
        prv = sc_ref[2]

        dsts = (nxt, prv)
        sgns = (-1, 1)
        col0s = (0, HN)

        def rows(j):
            return pl.ds(j * RH, RH)

        def hop_rdma(r, t, j):
            ss, rs = t % NS, (t + 1) % NS
            return pltpu.make_async_remote_copy(
                src_ref=comm_ref.at[r, ss, rows(j)],
                dst_ref=comm_ref.at[r, rs, rows(j)],
                send_sem=send_sems.at[r, ss, j],
                recv_sem=recv_sems.at[r, rs, j],
                device_id=(dsts[r],),
                device_id_type=pl.DeviceIdType.MESH,
            )

        send_pending = {}

        def start_send(r, t, j):
            key = (r, t % NS, j)
            prev = send_pending.get(key)
            if prev is not None:
                prev.wait_send()
            d = hop_rdma(r, t, j)
            d.start()
            send_pending[key] = d

        def partial_chunk(r, c):
            return lax.dot_general(
                x_ref[c], w_ref[:, col0s[r]:col0s[r] + HN],
                dimension_numbers=(((1,), (0,)), ((), ())),
                preferred_element_type=jnp.float32)

        def compute_local(r, s):
            c = jnp.mod(k + sgns[r] * (s + 1), N_DEV)
            local_ref[r, s % 2] = partial_chunk(r, c)

        for r in range(2):
            comm_ref[r, 0] = partial_chunk(r, k).astype(jnp.bfloat16)
            compute_local(r, 0)

        barrier_sem = pltpu.get_barrier_semaphore()
        for nbr in (nxt, prv):
            pl.semaphore_signal(barrier_sem, inc=1, device_id=(nbr,),
                                device_id_type=pl.DeviceIdType.MESH)
        pl.semaphore_wait(barrier_sem, 2)

        for j in range(SC):
            for r in range(2):
                start_send(r, 0, j)

        out_dma = [[None, None], [None, None]]

        for s in range(N_DEV - 1):
            rs_slot = (s + 1) % NS
            if s + 1 < N_DEV - 1:
                for r in range(2):
                    compute_local(r, s + 1)
            for j in range(SC):
                for r in range(2):
                    hop_rdma(r, s, j).wait_recv()
                    acc = (comm_ref[r, rs_slot, j * RH:(j + 1) * RH]
                           .astype(jnp.float32)
                           + local_ref[r, s % 2, j * RH:(j + 1) * RH])
                    if s < N_DEV - 2:
                        comm_ref[r, rs_slot, j * RH:(j + 1) * RH] = (
                            acc.astype(jnp.bfloat16))
                        start_send(r, s + 1, j)
                    else:
                        rr = jnp.maximum(acc, 0.0)
                        comm_ref[r, rs_slot, j * RH:(j + 1) * RH] = (
                            rr.astype(jnp.bfloat16))
                        start_send(r, N_DEV - 1, j)
                        ostage_ref[r, 0, j * RH:(j + 1) * RH] = rr
            if s == N_DEV - 2:
                for r in range(2):
                    c_own = jnp.mod(k - sgns[r], N_DEV)
                    odma = pltpu.make_async_copy(
                        ostage_ref.at[r, 0],
                        out_ref.at[pl.ds(c_own * CH, CH), pl.ds(col0s[r], HN)],
                        out_sems.at[r, 0])
                    odma.start()
                    out_dma[r][0] = odma

        for h in range(N_DEV - 1):
            t = N_DEV - 1 + h
            rs_slot = (t + 1) % NS
            for j in range(SC):
                for r in range(2):
                    hop_rdma(r, t, j).wait_recv()
                    if h < N_DEV - 2:
                        start_send(r, t + 1, j)
            oslot = h % 2
            for r in range(2):
                c = jnp.mod(k + sgns[r] * h, N_DEV)
                if out_dma[r][oslot] is not None:
                    out_dma[r][oslot].wait()
                ostage_ref[r, oslot] = comm_ref[r, rs_slot].astype(jnp.float32)
                odma = pltpu.make_async_copy(
                    ostage_ref.at[r, oslot],
                    out_ref.at[pl.ds(c * CH, CH), pl.ds(col0s[r], HN)],
                    out_sems.at[r, oslot])
                odma.start()
                out_dma[r][oslot] = odma

        for r in range(2):
            for sl in range(2):
                out_dma[r][sl].wait()
        for d in send_pending.values():
            d.wait_send()

        @functools.partial(pl.run_scoped, sem2=pltpu.SemaphoreType.REGULAR)
        def _(sem2):
            for nbr in (nxt, prv):
                pl.semaphore_signal(sem2, inc=1, device_id=(nbr,),
                                    device_id_type=pl.DeviceIdType.MESH)
            pl.semaphore_wait(sem2, 2)

    return pl.pallas_call(
        body,
        out_shape=jax.ShapeDtypeStruct((M, N), jnp.float32),
        in_specs=[
            pl.BlockSpec(memory_space=pltpu.SMEM),
            pl.BlockSpec(memory_space=pltpu.VMEM),
            pl.BlockSpec(memory_space=pltpu.VMEM),
        ],
        out_specs=pl.BlockSpec(memory_space=pl.ANY),
        scratch_shapes=[
            pltpu.VMEM((2, NS, CH, HN), jnp.bfloat16),
            pltpu.VMEM((2, 2, CH, HN), jnp.float32),
            pltpu.VMEM((2, 2, CH, HN), jnp.float32),
            pltpu.SemaphoreType.DMA((2, NS, SC)),
            pltpu.SemaphoreType.DMA((2, NS, SC)),
            pltpu.SemaphoreType.DMA((2, 2)),
        ],
        compiler_params=pltpu.CompilerParams(collective_id=0),
    )(scalars, x3, w)
